# baseline (speedup 1.0000x reference)
import functools

import jax
import jax.numpy as jnp
from jax import lax
from jax.experimental import pallas as pl
from jax.experimental.pallas import tpu as pltpu
from jax.experimental.pallas import tpu_sc as plsc

_SB = 8
_NB = 8


def _gather_kernel(batch, hist, embed_dim, num_workers):
    per_w = batch // num_workers
    n_chunks = per_w // _SB

    mesh = plsc.VectorSubcoreMesh(core_axis_name="c", subcore_axis_name="s")

    @functools.partial(
        pl.kernel,
        mesh=mesh,
        compiler_params=pltpu.CompilerParams(use_tc_tiling_on_sc=False),
        out_type=jax.ShapeDtypeStruct((batch, hist, embed_dim), jnp.float32),
        scratch_types=[
            pltpu.VMEM((per_w, hist), jnp.int32),
            pltpu.VMEM((_NB * _SB, hist, embed_dim), jnp.float32),
        ]
        + [pltpu.SemaphoreType.DMA] * (2 * _NB),
    )
    def k(idx_hbm, tab_hbm, out_hbm, idx_v, rows_v, *sems):
        gsem, wsem = sems[:_NB], sems[_NB:]
        wid = lax.axis_index("s") * 2 + lax.axis_index("c")
        batch_base = wid * per_w
        pltpu.sync_copy(idx_hbm.at[pl.ds(batch_base, per_w)], idx_v)

        def slot(b):
            return rows_v.at[pl.ds(b * _SB, _SB)]

        def fire_g(i, b):
            for j in range(_SB):
                pltpu.async_copy(
                    tab_hbm.at[idx_v.at[i * _SB + j]],
                    rows_v.at[b * _SB + j],
                    gsem[b],
                )

        def wait_g(b):
            pltpu.make_async_copy(
                out_hbm.at[pl.ds(batch_base, _SB)], slot(b), gsem[b]
            ).wait()

        def fire_w(i, b):
            pltpu.async_copy(
                slot(b), out_hbm.at[pl.ds(batch_base + i * _SB, _SB)], wsem[b]
            )

        def wait_w(b):
            pltpu.make_async_copy(
                slot(b), out_hbm.at[pl.ds(batch_base, _SB)], wsem[b]
            ).wait()

        for b in range(_NB - 1):
            fire_g(b, b)

        def body(g, carry):
            for b in range(_NB):
                i = g * _NB + b
                wait_g(b)
                fire_w(i, b)
                j = i + _NB - 1
                bj = (b - 1) % _NB

                @pl.when(j < n_chunks)
                def _():
                    @pl.when(j >= _NB)
                    def _():
                        wait_w(bj)

                    fire_g(j, bj)

            return carry

        lax.fori_loop(0, n_chunks // _NB, body, 0)
        for b in range(_NB):
            wait_w(b)

    return k


def kernel(batch_id, content):
    b, h = batch_id.shape
    v, d = content.shape
    return _gather_kernel(b, h, d, 32)(batch_id, content)


# speedup vs baseline: 1.0417x; 1.0417x over previous
import functools

import jax
import jax.numpy as jnp
from jax import lax
from jax.experimental import pallas as pl
from jax.experimental.pallas import tpu as pltpu
from jax.experimental.pallas import tpu_sc as plsc


_CB = 2048


def _repack_body(xt_ref, out_ref):
    x = xt_ref[...]
    y = jnp.transpose(x, (1, 0))
    out_ref[...] = jnp.concatenate([y, y], axis=1)


def _repack(content_t):
    d, v = content_t.shape
    grid = (v + _CB - 1) // _CB
    return pl.pallas_call(
        _repack_body,
        grid=(grid,),
        in_specs=[pl.BlockSpec((d, _CB), lambda i: (0, i))],
        out_specs=pl.BlockSpec((_CB, 2 * d), lambda i: (i, 0)),
        out_shape=jax.ShapeDtypeStruct((v, 2 * d), jnp.float32),
    )(content_t)


_SB = 4
_NB = 8


def _gather_kernel(batch, hist, embed_dim, num_workers):
    per_w = batch // num_workers
    n_chunks = per_w // _SB

    mesh = plsc.VectorSubcoreMesh(core_axis_name="c", subcore_axis_name="s")

    @functools.partial(
        pl.kernel,
        mesh=mesh,
        compiler_params=pltpu.CompilerParams(use_tc_tiling_on_sc=False),
        out_type=jax.ShapeDtypeStruct((batch, hist, embed_dim), jnp.float32),
        scratch_types=[
            pltpu.VMEM((per_w, hist), jnp.int32),
            pltpu.VMEM((_NB * _SB, hist, 2 * embed_dim), jnp.float32),
        ]
        + [pltpu.SemaphoreType.DMA] * (2 * _NB),
    )
    def k(idx_hbm, tab_hbm, out_hbm, idx_v, rows_v, *sems):
        gsem, wsem = sems[:_NB], sems[_NB:]
        wid = lax.axis_index("s") * 2 + lax.axis_index("c")
        batch_base = wid * per_w
        pltpu.sync_copy(idx_hbm.at[pl.ds(batch_base, per_w)], idx_v)

        def fire_g(i, b):
            for j in range(_SB):
                pltpu.async_copy(
                    tab_hbm.at[idx_v.at[i * _SB + j]],
                    rows_v.at[b * _SB + j],
                    gsem[b],
                )

        def wait_g(b):
            for j in range(_SB):
                pltpu.make_async_copy(
                    tab_hbm.at[pl.ds(0, hist)], rows_v.at[b * _SB + j], gsem[b]
                ).wait()

        def fire_w(i, b):
            pltpu.async_copy(
                rows_v.at[pl.ds(b * _SB, _SB), :, pl.ds(0, embed_dim)],
                out_hbm.at[pl.ds(batch_base + i * _SB, _SB)],
                wsem[b],
            )

        def wait_w(b):
            pltpu.make_async_copy(
                rows_v.at[pl.ds(b * _SB, _SB), :, pl.ds(0, embed_dim)],
                out_hbm.at[pl.ds(batch_base, _SB)],
                wsem[b],
            ).wait()

        for b in range(_NB - 1):
            fire_g(b, b)

        def body(g, carry):
            for b in range(_NB):
                i = g * _NB + b
                wait_g(b)
                fire_w(i, b)
                j = i + _NB - 1
                bj = (b - 1) % _NB

                @pl.when(j < n_chunks)
                def _():
                    @pl.when(j >= _NB)
                    def _():
                        wait_w(bj)

                    fire_g(j, bj)

            return carry

        lax.fori_loop(0, n_chunks // _NB, body, 0)
        for b in range(_NB):
            wait_w(b)

    return k


def kernel(batch_id, content):
    b, h = batch_id.shape
    v, d = content.shape
    tab = _repack(jnp.transpose(content))
    return _gather_kernel(b, h, d, 32)(batch_id, tab)


# speedup vs baseline: 1.2029x; 1.1548x over previous
import functools

import jax
import jax.numpy as jnp
from jax import lax
from jax.experimental import pallas as pl
from jax.experimental.pallas import tpu as pltpu
from jax.experimental.pallas import tpu_sc as plsc


_CB = 2048


def _repack_body(xt_ref, out_ref):
    x = xt_ref[...]
    y = jnp.transpose(x, (1, 0))
    out_ref[:, 0:64] = y
    out_ref[:, 64:128] = y


def _repack(content_t):
    d, v = content_t.shape
    grid = (v + _CB - 1) // _CB
    return pl.pallas_call(
        _repack_body,
        grid=(grid,),
        in_specs=[pl.BlockSpec((d, _CB), lambda i: (0, i))],
        out_specs=pl.BlockSpec((_CB, 2 * d), lambda i: (i, 0)),
        out_shape=jax.ShapeDtypeStruct((v, 2 * d), jnp.float32),
    )(content_t)


_SB = 4
_NB = 8
_HP = 24
_DP = 128


def _gather_kernel(batch, hist, embed_dim, num_workers):
    per_w = batch // num_workers
    n_chunks = per_w // _SB

    mesh = plsc.VectorSubcoreMesh(core_axis_name="c", subcore_axis_name="s")

    @functools.partial(
        pl.kernel,
        mesh=mesh,
        compiler_params=pltpu.CompilerParams(use_tc_tiling_on_sc=False),
        out_type=jax.ShapeDtypeStruct((batch, _HP, _DP), jnp.float32),
        scratch_types=[
            pltpu.VMEM((per_w, hist), jnp.int32),
            pltpu.VMEM((_NB * _SB, hist, 2 * embed_dim), jnp.float32),
        ]
        + [pltpu.SemaphoreType.DMA] * (2 * _NB),
    )
    def k(idx_hbm, tab_hbm, out_hbm, idx_v, rows_v, *sems):
        gsem, wsem = sems[:_NB], sems[_NB:]
        wid = lax.axis_index("s") * 2 + lax.axis_index("c")
        batch_base = wid * per_w
        pltpu.sync_copy(idx_hbm.at[pl.ds(batch_base, per_w)], idx_v)

        def fire_g(i, b):
            for j in range(_SB):
                pltpu.async_copy(
                    tab_hbm.at[idx_v.at[i * _SB + j]],
                    rows_v.at[b * _SB + j],
                    gsem[b],
                )

        def wait_g(b):
            for j in range(_SB):
                pltpu.make_async_copy(
                    tab_hbm.at[pl.ds(0, hist)], rows_v.at[b * _SB + j], gsem[b]
                ).wait()

        def fire_w(i, b):
            pltpu.async_copy(
                rows_v.at[pl.ds(b * _SB, _SB)],
                out_hbm.at[pl.ds(batch_base + i * _SB, _SB), pl.ds(0, hist)],
                wsem[b],
            )

        def wait_w(b):
            pltpu.make_async_copy(
                rows_v.at[pl.ds(b * _SB, _SB)],
                out_hbm.at[pl.ds(batch_base, _SB), pl.ds(0, hist)],
                wsem[b],
            ).wait()

        for b in range(_NB - 1):
            fire_g(b, b)

        def body(g, carry):
            for b in range(_NB):
                i = g * _NB + b
                wait_g(b)
                fire_w(i, b)
                j = i + _NB - 1
                bj = (b - 1) % _NB

                @pl.when(j < n_chunks)
                def _():
                    @pl.when(j >= _NB)
                    def _():
                        wait_w(bj)

                    fire_g(j, bj)

            return carry

        lax.fori_loop(0, n_chunks // _NB, body, 0)
        for b in range(_NB):
            wait_w(b)

    return k


def kernel(batch_id, content):
    b, h = batch_id.shape
    v, d = content.shape
    tab = _repack(jnp.transpose(content))
    padded = _gather_kernel(b, h, d, 32)(batch_id, tab)
    return padded[:, :h, :d]


# speedup vs baseline: 1.2499x; 1.0391x over previous
import functools

import jax
import jax.numpy as jnp
from jax import lax
from jax.experimental import pallas as pl
from jax.experimental.pallas import tpu as pltpu
from jax.experimental.pallas import tpu_sc as plsc


_CB = 2048


def _repack_body(xt_ref, out_ref):
    x = xt_ref[...]
    y = jnp.transpose(x, (1, 0))
    out_ref[:, 0:64] = y
    out_ref[:, 64:128] = y


def _repack(content_t):
    d, v = content_t.shape
    grid = (v + _CB - 1) // _CB
    return pl.pallas_call(
        _repack_body,
        grid=(grid,),
        in_specs=[pl.BlockSpec((d, _CB), lambda i: (0, i))],
        out_specs=pl.BlockSpec((_CB, 2 * d), lambda i: (i, 0)),
        out_shape=jax.ShapeDtypeStruct((v, 2 * d), jnp.float32),
    )(content_t)


_SB = 4
_NB = 8
_HP = 24
_DP = 128


def _gather_kernel(batch, hist, embed_dim, num_workers):
    per_w = batch // num_workers
    n_chunks = per_w // _SB

    mesh = plsc.VectorSubcoreMesh(core_axis_name="c", subcore_axis_name="s")

    @functools.partial(
        pl.kernel,
        mesh=mesh,
        compiler_params=pltpu.CompilerParams(use_tc_tiling_on_sc=False),
        out_type=jax.ShapeDtypeStruct((batch, _HP, _DP), jnp.float32),
        scratch_types=[
            pltpu.VMEM((per_w, hist), jnp.int32),
            pltpu.VMEM((_NB * _SB, hist, 2 * embed_dim), jnp.float32),
        ]
        + [pltpu.SemaphoreType.DMA] * (2 * _NB),
    )
    def k(idx_hbm, tab_hbm, out_hbm, idx_v, rows_v, *sems):
        gsem, wsem = sems[:_NB], sems[_NB:]
        wid = lax.axis_index("s") * 2 + lax.axis_index("c")
        batch_base = wid * per_w
        pltpu.sync_copy(idx_hbm.at[pl.ds(batch_base, per_w)], idx_v)

        def fire_g(i, b):
            for j in range(_SB):
                pltpu.async_copy(
                    tab_hbm.at[idx_v.at[i * _SB + j]],
                    rows_v.at[b * _SB + j],
                    gsem[b],
                )

        def wait_g(b):
            for j in range(_SB):
                pltpu.make_async_copy(
                    tab_hbm.at[pl.ds(0, hist)], rows_v.at[b * _SB + j], gsem[b]
                ).wait()

        def fire_w(i, b):
            pltpu.async_copy(
                rows_v.at[pl.ds(b * _SB, _SB), :, pl.ds(0, embed_dim)],
                out_hbm.at[
                    pl.ds(batch_base + i * _SB, _SB),
                    pl.ds(0, hist),
                    pl.ds(0, embed_dim),
                ],
                wsem[b],
            )

        def wait_w(b):
            pltpu.make_async_copy(
                rows_v.at[pl.ds(b * _SB, _SB), :, pl.ds(0, embed_dim)],
                out_hbm.at[
                    pl.ds(batch_base, _SB), pl.ds(0, hist), pl.ds(0, embed_dim)
                ],
                wsem[b],
            ).wait()

        for b in range(_NB - 1):
            fire_g(b, b)

        def body(g, carry):
            for b in range(_NB):
                i = g * _NB + b
                wait_g(b)
                fire_w(i, b)
                j = i + _NB - 1
                bj = (b - 1) % _NB

                @pl.when(j < n_chunks)
                def _():
                    @pl.when(j >= _NB)
                    def _():
                        wait_w(bj)

                    fire_g(j, bj)

            return carry

        lax.fori_loop(0, n_chunks // _NB, body, 0)
        for b in range(_NB):
            wait_w(b)

    return k


def kernel(batch_id, content):
    b, h = batch_id.shape
    v, d = content.shape
    tab = _repack(jnp.transpose(content))
    padded = _gather_kernel(b, h, d, 32)(batch_id, tab)
    return padded[:, :h, :d]


# speedup vs baseline: 1.5147x; 1.2119x over previous
import functools

import jax
import jax.numpy as jnp
from jax import lax
from jax.experimental import pallas as pl
from jax.experimental.pallas import tpu as pltpu
from jax.experimental.pallas import tpu_sc as plsc


_CB = 4096


def _repack_body(xt_ref, out_ref):
    x = xt_ref[...]
    y = jnp.transpose(x, (1, 0))
    out_ref[:, 0:64] = y
    out_ref[:, 64:128] = y


def _repack(content_t):
    d, v = content_t.shape
    grid = (v + _CB - 1) // _CB
    return pl.pallas_call(
        _repack_body,
        grid=(grid,),
        in_specs=[pl.BlockSpec((d, _CB), lambda i: (0, i))],
        out_specs=pl.BlockSpec((_CB, 2 * d), lambda i: (i, 0)),
        out_shape=jax.ShapeDtypeStruct((v, 2 * d), jnp.float32),
    )(content_t)


_SB = 4
_NB = 8
_HP = 24
_DP = 128


def _gather_kernel(batch, hist, embed_dim, num_workers):
    per_w = batch // num_workers
    n_chunks = per_w // _SB

    mesh = plsc.VectorSubcoreMesh(core_axis_name="c", subcore_axis_name="s")

    @functools.partial(
        pl.kernel,
        mesh=mesh,
        compiler_params=pltpu.CompilerParams(use_tc_tiling_on_sc=False),
        out_type=jax.ShapeDtypeStruct((batch, _HP, _DP), jnp.float32),
        scratch_types=[
            pltpu.VMEM((per_w, hist), jnp.int32),
            pltpu.VMEM((_NB * _SB, hist, 2 * embed_dim), jnp.float32),
        ]
        + [pltpu.SemaphoreType.DMA] * (2 * _NB),
    )
    def k(idx_hbm, tab_hbm, out_hbm, idx_v, rows_v, *sems):
        gsem, wsem = sems[:_NB], sems[_NB:]
        wid = lax.axis_index("s") * 2 + lax.axis_index("c")
        batch_base = wid * per_w
        pltpu.sync_copy(idx_hbm.at[pl.ds(batch_base, per_w)], idx_v)

        def fire_g(i, b):
            for j in range(_SB):
                pltpu.async_copy(
                    tab_hbm.at[idx_v.at[i * _SB + j]],
                    rows_v.at[b * _SB + j],
                    gsem[b],
                )

        def wait_g(b):
            for j in range(_SB):
                pltpu.make_async_copy(
                    tab_hbm.at[pl.ds(0, hist)], rows_v.at[b * _SB + j], gsem[b]
                ).wait()

        def fire_w(i, b):
            pltpu.async_copy(
                rows_v.at[pl.ds(b * _SB, _SB), :, pl.ds(0, embed_dim)],
                out_hbm.at[
                    pl.ds(batch_base + i * _SB, _SB),
                    pl.ds(0, hist),
                    pl.ds(0, embed_dim),
                ],
                wsem[b],
            )

        def wait_w(b):
            pltpu.make_async_copy(
                rows_v.at[pl.ds(b * _SB, _SB), :, pl.ds(0, embed_dim)],
                out_hbm.at[
                    pl.ds(batch_base, _SB), pl.ds(0, hist), pl.ds(0, embed_dim)
                ],
                wsem[b],
            ).wait()

        for b in range(_NB - 1):
            fire_g(b, b)

        def body(g, carry):
            for b in range(_NB):
                i = g * _NB + b
                wait_g(b)
                fire_w(i, b)
                j = i + _NB - 1
                bj = (b - 1) % _NB

                @pl.when(j < n_chunks)
                def _():
                    @pl.when(j >= _NB)
                    def _():
                        wait_w(bj)

                    fire_g(j, bj)

            return carry

        lax.fori_loop(0, n_chunks // _NB, body, 0)
        for b in range(_NB):
            wait_w(b)

    return k


def kernel(batch_id, content):
    b, h = batch_id.shape
    v, d = content.shape
    tab = _repack(jnp.transpose(content))
    padded = _gather_kernel(b, h, d, 32)(batch_id, tab)
    return padded[:, :h, :d]


# speedup vs baseline: 1.7200x; 1.1355x over previous
import functools

import jax
import jax.numpy as jnp
from jax import lax
from jax.experimental import pallas as pl
from jax.experimental.pallas import tpu as pltpu
from jax.experimental.pallas import tpu_sc as plsc


_CB = 8192


def _repack_body(xt_ref, out_ref):
    x = xt_ref[...]
    y = jnp.transpose(x, (1, 0))
    out_ref[:, 0:64] = y
    out_ref[:, 64:128] = y


def _repack(content_t):
    d, v = content_t.shape
    grid = (v + _CB - 1) // _CB
    return pl.pallas_call(
        _repack_body,
        grid=(grid,),
        in_specs=[pl.BlockSpec((d, _CB), lambda i: (0, i))],
        out_specs=pl.BlockSpec((_CB, 2 * d), lambda i: (i, 0)),
        out_shape=jax.ShapeDtypeStruct((v, 2 * d), jnp.float32),
    )(content_t)


_SB = 4
_NB = 8
_HP = 24
_DP = 128


def _gather_kernel(batch, hist, embed_dim, num_workers):
    per_w = batch // num_workers
    n_chunks = per_w // _SB

    mesh = plsc.VectorSubcoreMesh(core_axis_name="c", subcore_axis_name="s")

    @functools.partial(
        pl.kernel,
        mesh=mesh,
        compiler_params=pltpu.CompilerParams(use_tc_tiling_on_sc=False),
        out_type=jax.ShapeDtypeStruct((batch, _HP, _DP), jnp.float32),
        scratch_types=[
            pltpu.VMEM((per_w, hist), jnp.int32),
            pltpu.VMEM((_NB * _SB, hist, 2 * embed_dim), jnp.float32),
        ]
        + [pltpu.SemaphoreType.DMA] * (2 * _NB),
    )
    def k(idx_hbm, tab_hbm, out_hbm, idx_v, rows_v, *sems):
        gsem, wsem = sems[:_NB], sems[_NB:]
        wid = lax.axis_index("s") * 2 + lax.axis_index("c")
        batch_base = wid * per_w
        pltpu.sync_copy(idx_hbm.at[pl.ds(batch_base, per_w)], idx_v)

        def fire_g(i, b):
            for j in range(_SB):
                pltpu.async_copy(
                    tab_hbm.at[idx_v.at[i * _SB + j]],
                    rows_v.at[b * _SB + j],
                    gsem[b],
                )

        def wait_g(b):
            for j in range(_SB):
                pltpu.make_async_copy(
                    tab_hbm.at[pl.ds(0, hist)], rows_v.at[b * _SB + j], gsem[b]
                ).wait()

        def fire_w(i, b):
            pltpu.async_copy(
                rows_v.at[pl.ds(b * _SB, _SB), :, pl.ds(0, embed_dim)],
                out_hbm.at[
                    pl.ds(batch_base + i * _SB, _SB),
                    pl.ds(0, hist),
                    pl.ds(0, embed_dim),
                ],
                wsem[b],
            )

        def wait_w(b):
            pltpu.make_async_copy(
                rows_v.at[pl.ds(b * _SB, _SB), :, pl.ds(0, embed_dim)],
                out_hbm.at[
                    pl.ds(batch_base, _SB), pl.ds(0, hist), pl.ds(0, embed_dim)
                ],
                wsem[b],
            ).wait()

        for b in range(_NB - 1):
            fire_g(b, b)

        def body(g, carry):
            for b in range(_NB):
                i = g * _NB + b
                wait_g(b)
                fire_w(i, b)
                j = i + _NB - 1
                bj = (b - 1) % _NB

                @pl.when(j < n_chunks)
                def _():
                    @pl.when(j >= _NB)
                    def _():
                        wait_w(bj)

                    fire_g(j, bj)

            return carry

        lax.fori_loop(0, n_chunks // _NB, body, 0)
        for b in range(_NB):
            wait_w(b)

    return k


def kernel(batch_id, content):
    b, h = batch_id.shape
    v, d = content.shape
    tab = _repack(jnp.transpose(content))
    padded = _gather_kernel(b, h, d, 32)(batch_id, tab)
    return padded[:, :h, :d]


# speedup vs baseline: 1.8410x; 1.0703x over previous
import functools

import jax
import jax.numpy as jnp
from jax import lax
from jax.experimental import pallas as pl
from jax.experimental.pallas import tpu as pltpu
from jax.experimental.pallas import tpu_sc as plsc


_CB = 16384


def _repack_body(xt_ref, out_ref):
    x = xt_ref[...]
    y = jnp.transpose(x, (1, 0))
    out_ref[:, 0:64] = y
    out_ref[:, 64:128] = y


def _repack(content_t):
    d, v = content_t.shape
    grid = (v + _CB - 1) // _CB
    return pl.pallas_call(
        _repack_body,
        grid=(grid,),
        in_specs=[pl.BlockSpec((d, _CB), lambda i: (0, i))],
        out_specs=pl.BlockSpec((_CB, 2 * d), lambda i: (i, 0)),
        out_shape=jax.ShapeDtypeStruct((v, 2 * d), jnp.float32),
    )(content_t)


_SB = 4
_NB = 8
_HP = 24
_DP = 128


def _gather_kernel(batch, hist, embed_dim, num_workers):
    per_w = batch // num_workers
    n_chunks = per_w // _SB

    mesh = plsc.VectorSubcoreMesh(core_axis_name="c", subcore_axis_name="s")

    @functools.partial(
        pl.kernel,
        mesh=mesh,
        compiler_params=pltpu.CompilerParams(use_tc_tiling_on_sc=False),
        out_type=jax.ShapeDtypeStruct((batch, _HP, _DP), jnp.float32),
        scratch_types=[
            pltpu.VMEM((per_w, hist), jnp.int32),
            pltpu.VMEM((_NB * _SB, hist, 2 * embed_dim), jnp.float32),
        ]
        + [pltpu.SemaphoreType.DMA] * (2 * _NB),
    )
    def k(idx_hbm, tab_hbm, out_hbm, idx_v, rows_v, *sems):
        gsem, wsem = sems[:_NB], sems[_NB:]
        wid = lax.axis_index("s") * 2 + lax.axis_index("c")
        batch_base = wid * per_w
        pltpu.sync_copy(idx_hbm.at[pl.ds(batch_base, per_w)], idx_v)

        def fire_g(i, b):
            for j in range(_SB):
                pltpu.async_copy(
                    tab_hbm.at[idx_v.at[i * _SB + j]],
                    rows_v.at[b * _SB + j],
                    gsem[b],
                )

        def wait_g(b):
            for j in range(_SB):
                pltpu.make_async_copy(
                    tab_hbm.at[pl.ds(0, hist)], rows_v.at[b * _SB + j], gsem[b]
                ).wait()

        def fire_w(i, b):
            pltpu.async_copy(
                rows_v.at[pl.ds(b * _SB, _SB), :, pl.ds(0, embed_dim)],
                out_hbm.at[
                    pl.ds(batch_base + i * _SB, _SB),
                    pl.ds(0, hist),
                    pl.ds(0, embed_dim),
                ],
                wsem[b],
            )

        def wait_w(b):
            pltpu.make_async_copy(
                rows_v.at[pl.ds(b * _SB, _SB), :, pl.ds(0, embed_dim)],
                out_hbm.at[
                    pl.ds(batch_base, _SB), pl.ds(0, hist), pl.ds(0, embed_dim)
                ],
                wsem[b],
            ).wait()

        for b in range(_NB - 1):
            fire_g(b, b)

        def body(g, carry):
            for b in range(_NB):
                i = g * _NB + b
                wait_g(b)
                fire_w(i, b)
                j = i + _NB - 1
                bj = (b - 1) % _NB

                @pl.when(j < n_chunks)
                def _():
                    @pl.when(j >= _NB)
                    def _():
                        wait_w(bj)

                    fire_g(j, bj)

            return carry

        lax.fori_loop(0, n_chunks // _NB, body, 0)
        for b in range(_NB):
            wait_w(b)

    return k


def kernel(batch_id, content):
    b, h = batch_id.shape
    v, d = content.shape
    tab = _repack(jnp.transpose(content))
    padded = _gather_kernel(b, h, d, 32)(batch_id, tab)
    return padded[:, :h, :d]


# speedup vs baseline: 1.8817x; 1.0221x over previous
import functools

import jax
import jax.numpy as jnp
from jax import lax
from jax.experimental import pallas as pl
from jax.experimental.pallas import tpu as pltpu
from jax.experimental.pallas import tpu_sc as plsc


_CB = 24576


def _repack_body(xt_ref, out_ref):
    x = xt_ref[...]
    y = jnp.transpose(x, (1, 0))
    out_ref[:, 0:64] = y
    out_ref[:, 64:128] = y


def _repack(content_t):
    d, v = content_t.shape
    grid = (v + _CB - 1) // _CB
    return pl.pallas_call(
        _repack_body,
        grid=(grid,),
        in_specs=[pl.BlockSpec((d, _CB), lambda i: (0, i))],
        out_specs=pl.BlockSpec((_CB, 2 * d), lambda i: (i, 0)),
        out_shape=jax.ShapeDtypeStruct((v, 2 * d), jnp.float32),
    )(content_t)


_SB = 4
_NB = 8
_HP = 24
_DP = 128


def _gather_kernel(batch, hist, embed_dim, num_workers):
    per_w = batch // num_workers
    n_chunks = per_w // _SB

    mesh = plsc.VectorSubcoreMesh(core_axis_name="c", subcore_axis_name="s")

    @functools.partial(
        pl.kernel,
        mesh=mesh,
        compiler_params=pltpu.CompilerParams(use_tc_tiling_on_sc=False),
        out_type=jax.ShapeDtypeStruct((batch, _HP, _DP), jnp.float32),
        scratch_types=[
            pltpu.VMEM((per_w, hist), jnp.int32),
            pltpu.VMEM((_NB * _SB, hist, 2 * embed_dim), jnp.float32),
        ]
        + [pltpu.SemaphoreType.DMA] * (2 * _NB),
    )
    def k(idx_hbm, tab_hbm, out_hbm, idx_v, rows_v, *sems):
        gsem, wsem = sems[:_NB], sems[_NB:]
        wid = lax.axis_index("s") * 2 + lax.axis_index("c")
        batch_base = wid * per_w
        pltpu.sync_copy(idx_hbm.at[pl.ds(batch_base, per_w)], idx_v)

        def fire_g(i, b):
            for j in range(_SB):
                pltpu.async_copy(
                    tab_hbm.at[idx_v.at[i * _SB + j]],
                    rows_v.at[b * _SB + j],
                    gsem[b],
                )

        def wait_g(b):
            for j in range(_SB):
                pltpu.make_async_copy(
                    tab_hbm.at[pl.ds(0, hist)], rows_v.at[b * _SB + j], gsem[b]
                ).wait()

        def fire_w(i, b):
            pltpu.async_copy(
                rows_v.at[pl.ds(b * _SB, _SB), :, pl.ds(0, embed_dim)],
                out_hbm.at[
                    pl.ds(batch_base + i * _SB, _SB),
                    pl.ds(0, hist),
                    pl.ds(0, embed_dim),
                ],
                wsem[b],
            )

        def wait_w(b):
            pltpu.make_async_copy(
                rows_v.at[pl.ds(b * _SB, _SB), :, pl.ds(0, embed_dim)],
                out_hbm.at[
                    pl.ds(batch_base, _SB), pl.ds(0, hist), pl.ds(0, embed_dim)
                ],
                wsem[b],
            ).wait()

        for b in range(_NB - 1):
            fire_g(b, b)

        def body(g, carry):
            for b in range(_NB):
                i = g * _NB + b
                wait_g(b)
                fire_w(i, b)
                j = i + _NB - 1
                bj = (b - 1) % _NB

                @pl.when(j < n_chunks)
                def _():
                    @pl.when(j >= _NB)
                    def _():
                        wait_w(bj)

                    fire_g(j, bj)

            return carry

        lax.fori_loop(0, n_chunks // _NB, body, 0)
        for b in range(_NB):
            wait_w(b)

    return k


def kernel(batch_id, content):
    b, h = batch_id.shape
    v, d = content.shape
    tab = _repack(jnp.transpose(content))
    padded = _gather_kernel(b, h, d, 32)(batch_id, tab)
    return padded[:, :h, :d]


# speedup vs baseline: 1.8828x; 1.0006x over previous
import functools

import jax
import jax.numpy as jnp
from jax import lax
from jax.experimental import pallas as pl
from jax.experimental.pallas import tpu as pltpu
from jax.experimental.pallas import tpu_sc as plsc


_CB = 24576


def _repack_body(xt_ref, out_ref):
    x = xt_ref[...]
    y = jnp.transpose(x, (1, 0))
    out_ref[:, 0:64] = y
    out_ref[:, 64:128] = y


def _repack(content_t):
    d, v = content_t.shape
    grid = (v + _CB - 1) // _CB
    return pl.pallas_call(
        _repack_body,
        grid=(grid,),
        in_specs=[pl.BlockSpec((d, _CB), lambda i: (0, i))],
        out_specs=pl.BlockSpec((_CB, 2 * d), lambda i: (i, 0)),
        out_shape=jax.ShapeDtypeStruct((v, 2 * d), jnp.float32),
    )(content_t)


_SB = 8
_NB = 4
_HP = 24
_DP = 128


def _gather_kernel(batch, hist, embed_dim, num_workers):
    per_w = batch // num_workers
    n_chunks = per_w // _SB

    mesh = plsc.VectorSubcoreMesh(core_axis_name="c", subcore_axis_name="s")

    @functools.partial(
        pl.kernel,
        mesh=mesh,
        compiler_params=pltpu.CompilerParams(use_tc_tiling_on_sc=False),
        out_type=jax.ShapeDtypeStruct((batch, _HP, _DP), jnp.float32),
        scratch_types=[
            pltpu.VMEM((per_w, hist), jnp.int32),
            pltpu.VMEM((_NB * _SB, hist, 2 * embed_dim), jnp.float32),
        ]
        + [pltpu.SemaphoreType.DMA] * (2 * _NB),
    )
    def k(idx_hbm, tab_hbm, out_hbm, idx_v, rows_v, *sems):
        gsem, wsem = sems[:_NB], sems[_NB:]
        wid = lax.axis_index("s") * 2 + lax.axis_index("c")
        batch_base = wid * per_w
        pltpu.sync_copy(idx_hbm.at[pl.ds(batch_base, per_w)], idx_v)

        def fire_g(i, b):
            for j in range(_SB):
                pltpu.async_copy(
                    tab_hbm.at[idx_v.at[i * _SB + j]],
                    rows_v.at[b * _SB + j],
                    gsem[b],
                )

        def wait_g(b):
            for j in range(_SB):
                pltpu.make_async_copy(
                    tab_hbm.at[pl.ds(0, hist)], rows_v.at[b * _SB + j], gsem[b]
                ).wait()

        def fire_w(i, b):
            pltpu.async_copy(
                rows_v.at[pl.ds(b * _SB, _SB), :, pl.ds(0, embed_dim)],
                out_hbm.at[
                    pl.ds(batch_base + i * _SB, _SB),
                    pl.ds(0, hist),
                    pl.ds(0, embed_dim),
                ],
                wsem[b],
            )

        def wait_w(b):
            pltpu.make_async_copy(
                rows_v.at[pl.ds(b * _SB, _SB), :, pl.ds(0, embed_dim)],
                out_hbm.at[
                    pl.ds(batch_base, _SB), pl.ds(0, hist), pl.ds(0, embed_dim)
                ],
                wsem[b],
            ).wait()

        for b in range(_NB - 1):
            fire_g(b, b)

        def body(g, carry):
            for b in range(_NB):
                i = g * _NB + b
                wait_g(b)
                fire_w(i, b)
                j = i + _NB - 1
                bj = (b - 1) % _NB

                @pl.when(j < n_chunks)
                def _():
                    @pl.when(j >= _NB)
                    def _():
                        wait_w(bj)

                    fire_g(j, bj)

            return carry

        lax.fori_loop(0, n_chunks // _NB, body, 0)
        for b in range(_NB):
            wait_w(b)

    return k


def kernel(batch_id, content):
    b, h = batch_id.shape
    v, d = content.shape
    tab = _repack(jnp.transpose(content))
    padded = _gather_kernel(b, h, d, 32)(batch_id, tab)
    return padded[:, :h, :d]
